# baseline (speedup 1.0000x reference)
import jax
import jax.numpy as jnp
import numpy as np
from jax.experimental import pallas as pl
from jax.experimental.pallas import tpu as pltpu

LANE = 128
BLOCK_ROWS = 1024
N_CLASSES = 5
OUTP = 32

_IDX = np.arange(16)[:, None] - 2 * np.arange(8)[None, :] + 3
_SEL = (np.arange(7)[None, None, :] == _IDX[:, :, None]).astype(np.float32)
_SS = np.einsum('yoi,xpj->ijyxop', _SEL, _SEL).reshape(49, 16 * 16 * 8 * 8)
_REP = np.repeat(np.eye(16, dtype=np.float32), 64, axis=1)


def _prep_kernel(w2_ref, sb_ref, f6w_ref, f6b_ref, f7w_ref, f7b_ref,
                 cw_ref, cb_ref, bw_ref, bb_ref, ss_ref, rep_ref,
                 m_ref, bc_ref, w6_ref, b6_ref, w7_ref, b7_ref,
                 wh_ref, bh_ref):
    m_ref[...] = jnp.dot(w2_ref[...].astype(jnp.bfloat16), ss_ref[...],
                         preferred_element_type=jnp.float32).astype(jnp.bfloat16)
    bc_ref[...] = jnp.dot(sb_ref[...], rep_ref[...],
                          preferred_element_type=jnp.float32)
    w6_ref[...] = jnp.zeros_like(w6_ref)
    w6_ref[:, :64] = f6w_ref[...].astype(jnp.bfloat16)
    b6_ref[...] = jnp.zeros_like(b6_ref)
    b6_ref[:, :64] = f6b_ref[...]
    w7_ref[...] = jnp.zeros_like(w7_ref)
    w7_ref[:64, :64] = f7w_ref[...].astype(jnp.bfloat16)
    b7_ref[...] = jnp.zeros_like(b7_ref)
    b7_ref[:, :64] = f7b_ref[...]
    wh_ref[...] = jnp.zeros_like(wh_ref)
    wh_ref[:64, :N_CLASSES] = cw_ref[...].astype(jnp.bfloat16)
    wh_ref[:64, N_CLASSES:5 * N_CLASSES] = bw_ref[...].astype(jnp.bfloat16)
    bh_ref[...] = jnp.zeros_like(bh_ref)
    bh_ref[:, :N_CLASSES] = cb_ref[...]
    bh_ref[:, N_CLASSES:5 * N_CLASSES] = bb_ref[...]


def _head_kernel(x_ref, m_ref, bc_ref, w6_ref, b6_ref, w7_ref, b7_ref,
                 wh_ref, bh_ref, cls_ref, box_ref):
    xb = x_ref[...]
    m_full = jnp.concatenate([m_ref[c] for c in range(16)], axis=-1)
    feat = jnp.dot(xb, m_full, preferred_element_type=jnp.float32)
    feat = jnp.maximum(feat + bc_ref[...], 0.0).astype(jnp.bfloat16)
    h = jnp.dot(feat, w6_ref[...], preferred_element_type=jnp.float32)
    h = jnp.maximum(h + b6_ref[...], 0.0).astype(jnp.bfloat16)
    h = jnp.dot(h, w7_ref[...], preferred_element_type=jnp.float32)
    h = jnp.maximum(h + b7_ref[...], 0.0).astype(jnp.bfloat16)
    out = (jnp.dot(h, wh_ref[...], preferred_element_type=jnp.float32)
           + bh_ref[...])
    cls_ref[...] = out[:, :N_CLASSES]
    box_ref[...] = out[:, N_CLASSES:N_CLASSES + 4 * N_CLASSES]


def _full(shape):
    return pl.BlockSpec(shape, lambda: tuple(0 for _ in shape))


def _prepare(stem_w, stem_b, fc6_w, fc6_b, fc7_w, fc7_b,
             cls_w, cls_b, box_w, box_b):
    outs = pl.pallas_call(
        _prep_kernel,
        out_shape=[
            jax.ShapeDtypeStruct((48, 16384), jnp.bfloat16),
            jax.ShapeDtypeStruct((1, 1024), jnp.float32),
            jax.ShapeDtypeStruct((1024, LANE), jnp.bfloat16),
            jax.ShapeDtypeStruct((1, LANE), jnp.float32),
            jax.ShapeDtypeStruct((LANE, LANE), jnp.bfloat16),
            jax.ShapeDtypeStruct((1, LANE), jnp.float32),
            jax.ShapeDtypeStruct((LANE, OUTP), jnp.bfloat16),
            jax.ShapeDtypeStruct((1, OUTP), jnp.float32),
        ],
        in_specs=[
            _full((48, 49)), _full((1, 16)),
            _full((1024, 64)), _full((1, 64)),
            _full((64, 64)), _full((1, 64)),
            _full((64, N_CLASSES)), _full((1, N_CLASSES)),
            _full((64, 4 * N_CLASSES)), _full((1, 4 * N_CLASSES)),
            _full((49, 16384)), _full((16, 1024)),
        ],
        out_specs=[
            _full((48, 16384)), _full((1, 1024)),
            _full((1024, LANE)), _full((1, LANE)),
            _full((LANE, LANE)), _full((1, LANE)),
            _full((LANE, OUTP)), _full((1, OUTP)),
        ],
    )(stem_w.reshape(48, 49), stem_b.reshape(1, 16),
      fc6_w, fc6_b.reshape(1, 64), fc7_w, fc7_b.reshape(1, 64),
      cls_w, cls_b.reshape(1, N_CLASSES), box_w, box_b.reshape(1, 4 * N_CLASSES),
      jnp.asarray(_SS, jnp.bfloat16), jnp.asarray(_REP))
    return outs


def kernel(stem_w, stem_b, fc6_w, fc6_b, fc7_w, fc7_b,
           cls_w, cls_b, box_w, box_b, x):
    B = x.shape[0]
    br = min(BLOCK_ROWS, B)

    m48, bc, w6, b6, w7, b7, wh, bh = _prepare(
        stem_w, stem_b, fc6_w, fc6_b, fc7_w, fc7_b,
        cls_w, cls_b, box_w, box_b)
    m = m48.reshape(16, 3, 16, 16, 8, 8).reshape(16, 768, 64)

    xf = x.reshape(B, 768).astype(jnp.bfloat16)

    cls_out, box_out = pl.pallas_call(
        _head_kernel,
        out_shape=[jax.ShapeDtypeStruct((B, N_CLASSES), jnp.float32),
                   jax.ShapeDtypeStruct((B, 4 * N_CLASSES), jnp.float32)],
        grid=(B // br,),
        in_specs=[
            pl.BlockSpec((br, 768), lambda i: (i, 0)),
            pl.BlockSpec((16, 768, 64), lambda i: (0, 0, 0)),
            pl.BlockSpec((1, 1024), lambda i: (0, 0)),
            pl.BlockSpec((1024, LANE), lambda i: (0, 0)),
            pl.BlockSpec((1, LANE), lambda i: (0, 0)),
            pl.BlockSpec((LANE, LANE), lambda i: (0, 0)),
            pl.BlockSpec((1, LANE), lambda i: (0, 0)),
            pl.BlockSpec((LANE, OUTP), lambda i: (0, 0)),
            pl.BlockSpec((1, OUTP), lambda i: (0, 0)),
        ],
        out_specs=[pl.BlockSpec((br, N_CLASSES), lambda i: (i, 0)),
                   pl.BlockSpec((br, 4 * N_CLASSES), lambda i: (i, 0))],
        compiler_params=pltpu.CompilerParams(
            dimension_semantics=("parallel",),
        ),
    )(xf, m, bc, w6, b6, w7, b7, wh, bh)

    return {"class_logits": cls_out, "box_regression": box_out}


# speedup vs baseline: 1.0407x; 1.0407x over previous
import jax
import jax.numpy as jnp
import numpy as np
from jax.experimental import pallas as pl
from jax.experimental.pallas import tpu as pltpu

LANE = 128
BLOCK_ROWS = 1024
N_CLASSES = 5
OUTP = 32

_IDX = np.arange(16)[:, None] - 2 * np.arange(8)[None, :] + 3
_SEL = (np.arange(7)[None, None, :] == _IDX[:, :, None]).astype(np.float32)
_SS = np.einsum('yoi,xpj->ijyxop', _SEL, _SEL).reshape(49, 16 * 16 * 8 * 8)


def _head_kernel(x_ref, m_ref, bc_ref, w6_ref, b6_ref, w7_ref, b7_ref,
                 wh_ref, bh_ref, cls_ref, box_ref):
    xb = x_ref[...]
    m_full = jnp.concatenate([m_ref[c] for c in range(16)], axis=-1)
    feat = jnp.dot(xb, m_full, preferred_element_type=jnp.float32)
    feat = jnp.maximum(feat + bc_ref[...], 0.0).astype(jnp.bfloat16)
    h = jnp.dot(feat, w6_ref[...], preferred_element_type=jnp.float32)
    h = jnp.maximum(h + b6_ref[...], 0.0).astype(jnp.bfloat16)
    h = jnp.dot(h, w7_ref[...], preferred_element_type=jnp.float32)
    h = jnp.maximum(h + b7_ref[...], 0.0).astype(jnp.bfloat16)
    out = (jnp.dot(h, wh_ref[...], preferred_element_type=jnp.float32)
           + bh_ref[...])
    cls_ref[...] = out[:, :N_CLASSES]
    box_ref[...] = out[:, N_CLASSES:N_CLASSES + 4 * N_CLASSES]


def _conv_as_dense(stem_w):
    m = jnp.dot(stem_w.reshape(48, 49), jnp.asarray(_SS))
    return m.reshape(16, 3, 16, 16, 8, 8).reshape(16, 768, 64)


def kernel(stem_w, stem_b, fc6_w, fc6_b, fc7_w, fc7_b,
           cls_w, cls_b, box_w, box_b, x):
    B = x.shape[0]
    br = min(BLOCK_ROWS, B)
    pad = LANE - 64

    m = _conv_as_dense(stem_w).astype(jnp.bfloat16)
    bc = jnp.repeat(stem_b, 64)[None, :].astype(jnp.float32)
    w6 = jnp.pad(fc6_w, ((0, 0), (0, pad))).astype(jnp.bfloat16)
    b6 = jnp.pad(fc6_b, (0, pad))[None, :].astype(jnp.float32)
    w7 = jnp.pad(fc7_w, ((0, pad), (0, pad))).astype(jnp.bfloat16)
    b7 = jnp.pad(fc7_b, (0, pad))[None, :].astype(jnp.float32)
    wh = jnp.concatenate([cls_w, box_w], axis=1)
    n_out = wh.shape[1]
    wh = jnp.pad(wh, ((0, pad), (0, OUTP - n_out))).astype(jnp.bfloat16)
    bh = jnp.pad(jnp.concatenate([cls_b, box_b]),
                 (0, OUTP - n_out))[None, :].astype(jnp.float32)

    xf = x.reshape(B, 768).astype(jnp.bfloat16)

    cls_out, box_out = pl.pallas_call(
        _head_kernel,
        out_shape=[jax.ShapeDtypeStruct((B, N_CLASSES), jnp.float32),
                   jax.ShapeDtypeStruct((B, 4 * N_CLASSES), jnp.float32)],
        grid=(B // br,),
        in_specs=[
            pl.BlockSpec((br, 768), lambda i: (i, 0)),
            pl.BlockSpec((16, 768, 64), lambda i: (0, 0, 0)),
            pl.BlockSpec((1, 1024), lambda i: (0, 0)),
            pl.BlockSpec((1024, LANE), lambda i: (0, 0)),
            pl.BlockSpec((1, LANE), lambda i: (0, 0)),
            pl.BlockSpec((LANE, LANE), lambda i: (0, 0)),
            pl.BlockSpec((1, LANE), lambda i: (0, 0)),
            pl.BlockSpec((LANE, OUTP), lambda i: (0, 0)),
            pl.BlockSpec((1, OUTP), lambda i: (0, 0)),
        ],
        out_specs=[pl.BlockSpec((br, N_CLASSES), lambda i: (i, 0)),
                   pl.BlockSpec((br, 4 * N_CLASSES), lambda i: (i, 0))],
        compiler_params=pltpu.CompilerParams(
            dimension_semantics=("parallel",),
        ),
    )(xf, m, bc, w6, b6, w7, b7, wh, bh)

    return {"class_logits": cls_out, "box_regression": box_out}
